# baseline (speedup 1.0000x reference)
import functools

import jax
import jax.numpy as jnp
from jax import lax
from jax.experimental import pallas as pl
from jax.experimental.pallas import tpu as pltpu
from jax.experimental.pallas import tpu_sc as plsc

ALPHA = 5.0
EPS = 1e-8

_NW = 32
_BPW = 8
_SC_ROWS = _NW * _BPW
_CW = 1024


def _tc_partial_kernel(labels_ref, x_ref, out_ref, *, n_cols, block_rows):
    x = x_ref[...]
    ss = jnp.sum(x * x, axis=1, keepdims=True)
    lcol = jax.lax.broadcasted_iota(jnp.int32, (block_rows, n_cols), 1)
    g = jnp.sum(jnp.where(lcol == labels_ref[...], x, 0.0), axis=1,
                keepdims=True)
    s = g / jnp.maximum(jnp.sqrt(ss), EPS)
    out_ref[0, 0, 0] = jnp.sum((1.0 - s) / (1.0 + s) * ALPHA)


def _sc_rows_kernel(x_hbm, lblbc_hbm, out_hbm, lbl_v, buf, obuf,
                    *, n_cols, row0):
    wid = lax.axis_index("s") * 2 + lax.axis_index("c")
    base = row0 + wid * _BPW

    pltpu.sync_copy(lblbc_hbm.at[pl.ds(base, _BPW)], lbl_v)
    lane = lax.iota(jnp.int32, 16)
    lblbc = [lbl_v[r, pl.ds(0, 16)] for r in range(_BPW)]

    n_full = n_cols // _CW
    tail = n_cols - n_full * _CW

    zero = jnp.zeros((16,), jnp.float32)
    zeros2 = tuple(zero for _ in range(2 * _BPW))

    def do_chunk(c0, width, carry):
        def kbody(k, carry):
            cols = lane + (c0 + k * 16)
            new = []
            for r in range(_BPW):
                v = buf[r, pl.ds(k * 16, 16)]
                new.append(carry[2 * r] + v * v)
                new.append(carry[2 * r + 1]
                           + jnp.where(cols == lblbc[r], v, 0.0))
            return tuple(new)

        return lax.fori_loop(0, width // 16, kbody, carry)

    def chunk_body(c, carry):
        c0 = c * _CW
        pltpu.sync_copy(x_hbm.at[pl.ds(base, _BPW), pl.ds(c0, _CW)], buf)
        return do_chunk(c0, _CW, carry)

    carry = lax.fori_loop(0, n_full, chunk_body, zeros2)

    if tail:
        c0 = n_full * _CW
        pltpu.sync_copy(x_hbm.at[pl.ds(base, _BPW), pl.ds(c0, tail)],
                        buf.at[:, pl.ds(0, tail)])
        carry = do_chunk(c0, tail, carry)

    for r in range(_BPW):
        obuf[r, pl.ds(0, 16)] = carry[2 * r]
        obuf[r, pl.ds(16, 16)] = carry[2 * r + 1]
    pltpu.sync_copy(obuf, out_hbm.at[pl.ds(wid * _BPW, _BPW)])


def _combine_kernel(tcp_ref, sc_ref, tail_ref, lbl_ref, out_ref,
                    *, n_rows, col0, tail_cols):
    sso = sc_ref[...]
    ss = jnp.sum(sso[:, 0:16], axis=1, keepdims=True)
    g = jnp.sum(sso[:, 16:32], axis=1, keepdims=True)
    t = tail_ref[...]
    ss = ss + jnp.sum(t * t, axis=1, keepdims=True)
    lcol = col0 + jax.lax.broadcasted_iota(
        jnp.int32, (_SC_ROWS, tail_cols), 1)
    g = g + jnp.sum(jnp.where(lcol == lbl_ref[...], t, 0.0), axis=1,
                    keepdims=True)
    s = g / jnp.maximum(jnp.sqrt(ss), EPS)
    sc_sum = jnp.sum((1.0 - s) / (1.0 + s) * ALPHA)
    out_ref[0, 0] = (jnp.sum(tcp_ref[...]) + sc_sum) / n_rows


def kernel(logits, labels):
    n_rows, n_cols = logits.shape
    labels_i = labels.astype(jnp.int32)
    n_tc = n_rows - _SC_ROWS
    block_rows = 32
    n_blocks = n_tc // block_rows
    labels2 = labels_i.reshape(n_rows, 1)
    lblbc2d = jnp.broadcast_to(labels_i[:, None], (n_rows, 16))

    n_cols_sc = (n_cols // 128) * 128

    sc_out = pl.kernel(
        functools.partial(_sc_rows_kernel, n_cols=n_cols_sc, row0=n_tc),
        out_type=jax.ShapeDtypeStruct((_SC_ROWS, 32), jnp.float32),
        mesh=plsc.VectorSubcoreMesh(core_axis_name="c", subcore_axis_name="s"),
        scratch_types=[
            pltpu.VMEM((_BPW, 16), jnp.int32),
            pltpu.VMEM((_BPW, _CW), jnp.float32),
            pltpu.VMEM((_BPW, 32), jnp.float32),
        ],
    )(logits, lblbc2d)

    tc_partials = pl.pallas_call(
        functools.partial(
            _tc_partial_kernel, n_cols=n_cols, block_rows=block_rows),
        grid=(n_blocks,),
        in_specs=[
            pl.BlockSpec((block_rows, 1), lambda rb: (rb, 0)),
            pl.BlockSpec((block_rows, n_cols), lambda rb: (rb, 0)),
        ],
        out_specs=pl.BlockSpec(
            (1, 1, 1), lambda rb: (rb, 0, 0), memory_space=pltpu.SMEM),
        out_shape=jax.ShapeDtypeStruct((n_blocks, 1, 1), jnp.float32),
    )(labels2, logits)

    tail_x = logits[n_tc:, n_cols_sc:]
    lbl_sc = labels2[n_tc:]
    out = pl.pallas_call(
        functools.partial(_combine_kernel, n_rows=n_rows, col0=n_cols_sc,
                          tail_cols=n_cols - n_cols_sc),
        out_specs=pl.BlockSpec(memory_space=pltpu.SMEM),
        out_shape=jax.ShapeDtypeStruct((1, 1), jnp.float32),
    )(tc_partials, sc_out, tail_x, lbl_sc)
    return out[0, 0]


# speedup vs baseline: 1.0656x; 1.0656x over previous
import functools

import jax
import jax.numpy as jnp
from jax import lax
from jax.experimental import pallas as pl
from jax.experimental.pallas import tpu as pltpu
from jax.experimental.pallas import tpu_sc as plsc

ALPHA = 5.0
EPS = 1e-8

_NW = 32
_BPW = 4
_SC_ROWS = _NW * _BPW
_CW = 1024


def _tc_partial_kernel(labels_ref, x_ref, out_ref, *, n_cols, block_rows):
    x = x_ref[...]
    ss = jnp.sum(x * x, axis=1, keepdims=True)
    lcol = jax.lax.broadcasted_iota(jnp.int32, (block_rows, n_cols), 1)
    g = jnp.sum(jnp.where(lcol == labels_ref[...], x, 0.0), axis=1,
                keepdims=True)
    s = g / jnp.maximum(jnp.sqrt(ss), EPS)
    out_ref[0, 0, 0] = jnp.sum((1.0 - s) / (1.0 + s) * ALPHA)


def _sc_rows_kernel(x_hbm, lblbc_hbm, out_hbm, lbl_v, buf, obuf,
                    *, n_cols, row0):
    wid = lax.axis_index("s") * 2 + lax.axis_index("c")
    base = row0 + wid * _BPW

    pltpu.sync_copy(lblbc_hbm.at[pl.ds(base, _BPW)], lbl_v)
    lane = lax.iota(jnp.int32, 16)
    lblbc = [lbl_v[r, pl.ds(0, 16)] for r in range(_BPW)]

    n_full = n_cols // _CW
    tail = n_cols - n_full * _CW

    zero = jnp.zeros((16,), jnp.float32)
    zeros2 = tuple(zero for _ in range(2 * _BPW))

    def do_chunk(c0, width, carry):
        def kbody(k, carry):
            cols = lane + (c0 + k * 16)
            new = []
            for r in range(_BPW):
                v = buf[r, pl.ds(k * 16, 16)]
                new.append(carry[2 * r] + v * v)
                new.append(carry[2 * r + 1]
                           + jnp.where(cols == lblbc[r], v, 0.0))
            return tuple(new)

        return lax.fori_loop(0, width // 16, kbody, carry)

    def chunk_body(c, carry):
        c0 = c * _CW
        pltpu.sync_copy(x_hbm.at[pl.ds(base, _BPW), pl.ds(c0, _CW)], buf)
        return do_chunk(c0, _CW, carry)

    carry = lax.fori_loop(0, n_full, chunk_body, zeros2)

    if tail:
        c0 = n_full * _CW
        pltpu.sync_copy(x_hbm.at[pl.ds(base, _BPW), pl.ds(c0, tail)],
                        buf.at[:, pl.ds(0, tail)])
        carry = do_chunk(c0, tail, carry)

    for r in range(_BPW):
        obuf[r, pl.ds(0, 16)] = carry[2 * r]
        obuf[r, pl.ds(16, 16)] = carry[2 * r + 1]
    pltpu.sync_copy(obuf, out_hbm.at[pl.ds(wid * _BPW, _BPW)])


def _combine_kernel(tcp_ref, sc_ref, tail_ref, lbl_ref, out_ref,
                    *, n_rows, col0, tail_cols):
    sso = sc_ref[...]
    ss = jnp.sum(sso[:, 0:16], axis=1, keepdims=True)
    g = jnp.sum(sso[:, 16:32], axis=1, keepdims=True)
    t = tail_ref[...]
    ss = ss + jnp.sum(t * t, axis=1, keepdims=True)
    lcol = col0 + jax.lax.broadcasted_iota(
        jnp.int32, (_SC_ROWS, tail_cols), 1)
    g = g + jnp.sum(jnp.where(lcol == lbl_ref[...], t, 0.0), axis=1,
                    keepdims=True)
    s = g / jnp.maximum(jnp.sqrt(ss), EPS)
    sc_sum = jnp.sum((1.0 - s) / (1.0 + s) * ALPHA)
    out_ref[0, 0] = (jnp.sum(tcp_ref[...]) + sc_sum) / n_rows


def kernel(logits, labels):
    n_rows, n_cols = logits.shape
    labels_i = labels.astype(jnp.int32)
    n_tc = n_rows - _SC_ROWS
    block_rows = 32
    n_blocks = n_tc // block_rows
    labels2 = labels_i.reshape(n_rows, 1)
    lblbc2d = jnp.broadcast_to(labels_i[:, None], (n_rows, 16))

    n_cols_sc = (n_cols // 128) * 128

    sc_out = pl.kernel(
        functools.partial(_sc_rows_kernel, n_cols=n_cols_sc, row0=n_tc),
        out_type=jax.ShapeDtypeStruct((_SC_ROWS, 32), jnp.float32),
        mesh=plsc.VectorSubcoreMesh(core_axis_name="c", subcore_axis_name="s"),
        scratch_types=[
            pltpu.VMEM((_BPW, 16), jnp.int32),
            pltpu.VMEM((_BPW, _CW), jnp.float32),
            pltpu.VMEM((_BPW, 32), jnp.float32),
        ],
    )(logits, lblbc2d)

    tc_partials = pl.pallas_call(
        functools.partial(
            _tc_partial_kernel, n_cols=n_cols, block_rows=block_rows),
        grid=(n_blocks,),
        in_specs=[
            pl.BlockSpec((block_rows, 1), lambda rb: (rb, 0)),
            pl.BlockSpec((block_rows, n_cols), lambda rb: (rb, 0)),
        ],
        out_specs=pl.BlockSpec(
            (1, 1, 1), lambda rb: (rb, 0, 0), memory_space=pltpu.SMEM),
        out_shape=jax.ShapeDtypeStruct((n_blocks, 1, 1), jnp.float32),
    )(labels2, logits)

    tail_x = logits[n_tc:, n_cols_sc:]
    lbl_sc = labels2[n_tc:]
    out = pl.pallas_call(
        functools.partial(_combine_kernel, n_rows=n_rows, col0=n_cols_sc,
                          tail_cols=n_cols - n_cols_sc),
        out_specs=pl.BlockSpec(memory_space=pltpu.SMEM),
        out_shape=jax.ShapeDtypeStruct((1, 1), jnp.float32),
    )(tc_partials, sc_out, tail_x, lbl_sc)
    return out[0, 0]


# speedup vs baseline: 1.1493x; 1.0785x over previous
import functools

import jax
import jax.numpy as jnp
from jax.experimental import pallas as pl
from jax.experimental.pallas import tpu as pltpu

ALPHA = 5.0
EPS = 1e-8


def _cosine_loss_kernel(labels_ref, x_ref, out_ref, *, n_rows, n_cols,
                        block_rows):
    rb = pl.program_id(0)

    @pl.when(rb == 0)
    def _init():
        out_ref[0, 0] = 0.0

    x = x_ref[...]
    ss = jnp.sum(x * x, axis=1, keepdims=True)
    lcol = jax.lax.broadcasted_iota(jnp.int32, (block_rows, n_cols), 1)
    g = jnp.sum(jnp.where(lcol == labels_ref[...], x, 0.0), axis=1,
                keepdims=True)
    s = g / jnp.maximum(jnp.sqrt(ss), EPS)
    loss_terms = (1.0 - s) / (1.0 + s) * ALPHA
    out_ref[0, 0] += jnp.sum(loss_terms) / n_rows


def kernel(logits, labels):
    n_rows, n_cols = logits.shape
    block_rows = 64
    n_blocks = n_rows // block_rows
    labels2 = labels.astype(jnp.int32).reshape(n_rows, 1)

    out = pl.pallas_call(
        functools.partial(
            _cosine_loss_kernel, n_rows=n_rows, n_cols=n_cols,
            block_rows=block_rows),
        grid=(n_blocks,),
        in_specs=[
            pl.BlockSpec((block_rows, 1), lambda rb: (rb, 0)),
            pl.BlockSpec((block_rows, n_cols), lambda rb: (rb, 0)),
        ],
        out_specs=pl.BlockSpec(
            (1, 1), lambda rb: (0, 0), memory_space=pltpu.SMEM),
        out_shape=jax.ShapeDtypeStruct((1, 1), jnp.float32),
    )(labels2, logits)
    return out[0, 0]
